# baseline (speedup 1.0000x reference)
import jax
import jax.numpy as jnp
from jax import lax
from jax.experimental import pallas as pl
from jax.experimental.pallas import tpu as pltpu
from jax.experimental.pallas import tpu_sc as plsc

K_CODES = 1024
DIM = 64
BETA = 0.25

N_ROWS = 16384
HALF_IMGS = 8
HALF_ROWS = HALF_IMGS * 32 * 32
ROW_BLOCK = 2048
N_BLOCKS = HALF_ROWS // ROW_BLOCK
IDX_ROWS = ROW_BLOCK // 128

NC = 2
NS = 16
NW = NC * NS
CHUNK = 128
N_CHUNKS = HALF_ROWS // CHUNK
CH_PER_W = N_CHUNKS // NW


def _argmin_body(x_ref, e_ref, idx_ref, lsum_ref, acc_ref):
    i = pl.program_id(0)
    x = x_ref[...].reshape(ROW_BLOCK, DIM)
    e = e_ref[...]
    pv = jnp.sum(x * x, axis=1, keepdims=True)
    el = jnp.sum(e * e, axis=0, keepdims=True)
    s2 = jnp.dot(x + x, e, preferred_element_type=jnp.float32)
    dist = (pv + el) - s2
    rowmin = jnp.min(dist, axis=1, keepdims=True)
    cols = lax.broadcasted_iota(jnp.int32, dist.shape, 1).astype(jnp.float32)
    idxf = jnp.min(jnp.where(dist == rowmin, cols, float(K_CODES)), axis=1, keepdims=True)
    idx_ref[...] = idxf.astype(jnp.int32).reshape(IDX_ROWS, 128)

    @pl.when(i == 0)
    def _init():
        acc_ref[0] = 0.0

    acc_ref[0] += jnp.sum(rowmin)

    @pl.when(i == N_BLOCKS - 1)
    def _fini():
        lsum_ref[0, 0] = acc_ref[0]


def _distances_argmin(x4d, embedding):
    return pl.pallas_call(
        _argmin_body,
        grid=(N_BLOCKS,),
        in_specs=[
            pl.BlockSpec((2, 32, 32, DIM), lambda i: (i, 0, 0, 0)),
            pl.BlockSpec((DIM, K_CODES), lambda i: (0, 0)),
        ],
        out_specs=[
            pl.BlockSpec((IDX_ROWS, 128), lambda i: (i, 0)),
            pl.BlockSpec(memory_space=pltpu.SMEM),
        ],
        out_shape=[
            jax.ShapeDtypeStruct((N_CHUNKS, CHUNK), jnp.int32),
            jax.ShapeDtypeStruct((1, 1), jnp.float32),
        ],
        scratch_shapes=[pltpu.SMEM((1,), jnp.float32)],
    )(x4d, embedding)


def _gather_body(table_hbm, idx_hbm, out_hbm, idx_v, rows_v, sem, sem_out):
    wid = lax.axis_index("s") * NC + lax.axis_index("c")
    base = wid * CH_PER_W
    pltpu.sync_copy(idx_hbm.at[pl.ds(base, CH_PER_W)], idx_v)
    copies = [
        pltpu.async_copy(table_hbm.at[idx_v.at[j]], rows_v.at[j], sem)
        for j in range(CH_PER_W)
    ]
    outs = []
    for j in range(CH_PER_W):
        copies[j].wait()
        outs.append(pltpu.async_copy(rows_v.at[j], out_hbm.at[base + j], sem_out))
    for c in outs:
        c.wait()


def _sc_gather(table, idx2d):
    k = pl.kernel(
        _gather_body,
        mesh=plsc.VectorSubcoreMesh(core_axis_name="c", subcore_axis_name="s"),
        out_type=jax.ShapeDtypeStruct((N_CHUNKS, CHUNK, DIM), jnp.float32),
        scratch_types=[
            pltpu.VMEM((CH_PER_W, CHUNK), jnp.int32),
            pltpu.VMEM((CH_PER_W, CHUNK, DIM), jnp.float32),
            pltpu.SemaphoreType.DMA,
            pltpu.SemaphoreType.DMA,
        ],
        compiler_params=pltpu.CompilerParams(use_tc_tiling_on_sc=False),
    )
    return k(table, idx2d)


def kernel(x, embedding):
    table = embedding.T
    x0, x1 = x[:HALF_IMGS], x[HALF_IMGS:]
    idx0, a0 = _distances_argmin(x0, embedding)
    q0 = _sc_gather(table, idx0)
    idx1, a1 = _distances_argmin(x1, embedding)
    q1 = _sc_gather(table, idx1)
    quantized = jnp.concatenate(
        [q0.reshape(HALF_IMGS, 32, 32, DIM), q1.reshape(HALF_IMGS, 32, 32, DIM)],
        axis=0)
    loss = (a0[0, 0] + a1[0, 0]) * ((1.0 + BETA) / (N_ROWS * DIM))
    return quantized, loss


# speedup vs baseline: 1.1950x; 1.1950x over previous
import jax
import jax.numpy as jnp
from jax import lax
from jax.experimental import pallas as pl
from jax.experimental.pallas import tpu as pltpu
from jax.experimental.pallas import tpu_sc as plsc

K_CODES = 1024
DIM = 64
BETA = 0.25

N_ROWS = 16384
ROW_BLOCK = 2048
N_BLOCKS = N_ROWS // ROW_BLOCK
IDX_ROWS = ROW_BLOCK // 128

NC = 2
NS = 16
NW = NC * NS
CHUNK = 128
N_CHUNKS = N_ROWS // CHUNK
CH_PER_W = N_CHUNKS // NW


def _argmin_body(x_ref, e_ref, idx_ref, lsum_ref, acc_ref):
    i = pl.program_id(0)
    x = x_ref[...].reshape(ROW_BLOCK, DIM)
    e = e_ref[...]
    pv = jnp.sum(x * x, axis=1, keepdims=True)
    el = jnp.sum(e * e, axis=0, keepdims=True)
    s2 = jnp.dot(x + x, e, preferred_element_type=jnp.float32)
    dist = (pv + el) - s2
    rowmin = jnp.min(dist, axis=1, keepdims=True)
    cols = lax.broadcasted_iota(jnp.int32, dist.shape, 1).astype(jnp.float32)
    idxf = jnp.min(jnp.where(dist == rowmin, cols, float(K_CODES)), axis=1, keepdims=True)
    idx_ref[...] = idxf.astype(jnp.int32).reshape(IDX_ROWS, 128)

    @pl.when(i == 0)
    def _init():
        acc_ref[0] = 0.0

    acc_ref[0] += jnp.sum(rowmin)

    @pl.when(i == N_BLOCKS - 1)
    def _fini():
        lsum_ref[0, 0] = acc_ref[0] * ((1.0 + BETA) / (N_ROWS * DIM))


def _distances_argmin(x4d, embedding):
    return pl.pallas_call(
        _argmin_body,
        grid=(N_BLOCKS,),
        in_specs=[
            pl.BlockSpec((2, 32, 32, DIM), lambda i: (i, 0, 0, 0)),
            pl.BlockSpec((DIM, K_CODES), lambda i: (0, 0)),
        ],
        out_specs=[
            pl.BlockSpec((IDX_ROWS, 128), lambda i: (i, 0)),
            pl.BlockSpec(memory_space=pltpu.SMEM),
        ],
        out_shape=[
            jax.ShapeDtypeStruct((N_CHUNKS, CHUNK), jnp.int32),
            jax.ShapeDtypeStruct((1, 1), jnp.float32),
        ],
        scratch_shapes=[pltpu.SMEM((1,), jnp.float32)],
    )(x4d, embedding)


def _gather_body(table_hbm, idx_hbm, out_hbm, idx_v, rows_v, sem, sem_out):
    wid = lax.axis_index("s") * NC + lax.axis_index("c")
    base = wid * CH_PER_W
    pltpu.sync_copy(idx_hbm.at[pl.ds(base, CH_PER_W)], idx_v)
    copies = [
        pltpu.async_copy(table_hbm.at[idx_v.at[j]], rows_v.at[j], sem)
        for j in range(CH_PER_W)
    ]
    outs = []
    for j in range(CH_PER_W):
        copies[j].wait()
        outs.append(pltpu.async_copy(rows_v.at[j], out_hbm.at[base + j], sem_out))
    for c in outs:
        c.wait()


def _sc_gather(table, idx2d):
    k = pl.kernel(
        _gather_body,
        mesh=plsc.VectorSubcoreMesh(core_axis_name="c", subcore_axis_name="s"),
        out_type=jax.ShapeDtypeStruct((N_CHUNKS, CHUNK, DIM), jnp.float32),
        scratch_types=[
            pltpu.VMEM((CH_PER_W, CHUNK), jnp.int32),
            pltpu.VMEM((CH_PER_W, CHUNK, DIM), jnp.float32),
            pltpu.SemaphoreType.DMA,
            pltpu.SemaphoreType.DMA,
        ],
        compiler_params=pltpu.CompilerParams(use_tc_tiling_on_sc=False),
    )
    return k(table, idx2d)


def kernel(x, embedding):
    table = embedding.T
    idx2d, lsum = _distances_argmin(x, embedding)
    q = _sc_gather(table, idx2d)
    quantized = q.reshape(x.shape)
    loss = lsum.reshape(())
    return quantized, loss


# speedup vs baseline: 1.2169x; 1.0183x over previous
import jax
import jax.numpy as jnp
from jax import lax
from jax.experimental import pallas as pl
from jax.experimental.pallas import tpu as pltpu
from jax.experimental.pallas import tpu_sc as plsc

K_CODES = 1024
DIM = 64
BETA = 0.25

N_ROWS = 16384
ROW_BLOCK = 2048
N_BLOCKS = N_ROWS // ROW_BLOCK
IDX_ROWS = ROW_BLOCK // 128

NC = 2
NS = 16
NW = NC * NS
CHUNK = 128
N_CHUNKS = N_ROWS // CHUNK
CH_PER_W = N_CHUNKS // NW


def _argmin_body(x_ref, e_ref, idx_ref, lsum_ref, acc_ref):
    i = pl.program_id(0)
    x = x_ref[...].reshape(ROW_BLOCK, DIM)
    e = e_ref[...]
    pv = jnp.sum(x * x, axis=1, keepdims=True)
    el = jnp.sum(e * e, axis=0, keepdims=True)
    s2 = jnp.dot(x + x, e, preferred_element_type=jnp.float32)
    dist = (pv + el) - s2
    rowmin = jnp.min(dist, axis=1, keepdims=True)
    cols = lax.broadcasted_iota(jnp.int32, dist.shape, 1).astype(jnp.float32)
    idxf = jnp.min(jnp.where(dist == rowmin, cols, float(K_CODES)), axis=1, keepdims=True)
    idx_ref[...] = idxf.astype(jnp.int32).reshape(IDX_ROWS, 128)

    @pl.when(i == 0)
    def _init():
        acc_ref[0] = 0.0

    acc_ref[0] += jnp.sum(rowmin)

    @pl.when(i == N_BLOCKS - 1)
    def _fini():
        lsum_ref[0, 0] = acc_ref[0] * ((1.0 + BETA) / (N_ROWS * DIM))


def _distances_argmin(x4d, embedding):
    return pl.pallas_call(
        _argmin_body,
        grid=(N_BLOCKS,),
        in_specs=[
            pl.BlockSpec((2, 32, 32, DIM), lambda i: (i, 0, 0, 0)),
            pl.BlockSpec((DIM, K_CODES), lambda i: (0, 0)),
        ],
        out_specs=[
            pl.BlockSpec((IDX_ROWS, 128), lambda i: (i, 0)),
            pl.BlockSpec(memory_space=pltpu.SMEM),
        ],
        out_shape=[
            jax.ShapeDtypeStruct((N_CHUNKS, CHUNK), jnp.int32),
            jax.ShapeDtypeStruct((1, 1), jnp.float32),
        ],
        scratch_shapes=[pltpu.SMEM((1,), jnp.float32)],
    )(x4d, embedding)


def _gather_body(table_hbm, idx_hbm, out_hbm, idx_v, rows_v, sem, sem_out):
    wid = lax.axis_index("s") * NC + lax.axis_index("c")
    base = wid * CH_PER_W
    pltpu.sync_copy(idx_hbm.at[pl.ds(base, CH_PER_W)], idx_v)
    copies = [
        pltpu.async_copy(table_hbm.at[idx_v.at[j]], rows_v.at[j], sem)
        for j in range(CH_PER_W)
    ]
    for c in copies:
        c.wait()
    pltpu.sync_copy(rows_v, out_hbm.at[pl.ds(base, CH_PER_W)])


def _sc_gather(table, idx2d):
    k = pl.kernel(
        _gather_body,
        mesh=plsc.VectorSubcoreMesh(core_axis_name="c", subcore_axis_name="s"),
        out_type=jax.ShapeDtypeStruct((N_CHUNKS, CHUNK, DIM), jnp.float32),
        scratch_types=[
            pltpu.VMEM((CH_PER_W, CHUNK), jnp.int32),
            pltpu.VMEM((CH_PER_W, CHUNK, DIM), jnp.float32),
            pltpu.SemaphoreType.DMA,
            pltpu.SemaphoreType.DMA,
        ],
        compiler_params=pltpu.CompilerParams(use_tc_tiling_on_sc=False),
    )
    return k(table, idx2d)


def kernel(x, embedding):
    table = embedding.T
    idx2d, lsum = _distances_argmin(x, embedding)
    q = _sc_gather(table, idx2d)
    quantized = q.reshape(x.shape)
    loss = lsum.reshape(())
    return quantized, loss


# speedup vs baseline: 1.2298x; 1.0106x over previous
import jax
import jax.numpy as jnp
from jax import lax
from jax.experimental import pallas as pl
from jax.experimental.pallas import tpu as pltpu
from jax.experimental.pallas import tpu_sc as plsc

K_CODES = 1024
DIM = 64
BETA = 0.25

N_ROWS = 16384
ROW_BLOCK = 4096
N_BLOCKS = N_ROWS // ROW_BLOCK
IDX_ROWS = ROW_BLOCK // 128

NC = 2
NS = 16
NW = NC * NS
CHUNK = 128
N_CHUNKS = N_ROWS // CHUNK
CH_PER_W = N_CHUNKS // NW


def _argmin_body(x_ref, e_ref, idx_ref, lsum_ref, acc_ref):
    i = pl.program_id(0)
    x = x_ref[...].reshape(ROW_BLOCK, DIM)
    e = e_ref[...]
    pv = jnp.sum(x * x, axis=1, keepdims=True)
    el = jnp.sum(e * e, axis=0, keepdims=True)
    s2 = jnp.dot(x + x, e, preferred_element_type=jnp.float32)
    dist = (pv + el) - s2
    rowmin = jnp.min(dist, axis=1, keepdims=True)
    cols = lax.broadcasted_iota(jnp.int32, dist.shape, 1).astype(jnp.float32)
    idxf = jnp.min(jnp.where(dist == rowmin, cols, float(K_CODES)), axis=1, keepdims=True)
    idx_ref[...] = idxf.astype(jnp.int32).reshape(IDX_ROWS, 128)

    @pl.when(i == 0)
    def _init():
        acc_ref[0] = 0.0

    acc_ref[0] += jnp.sum(rowmin)

    @pl.when(i == N_BLOCKS - 1)
    def _fini():
        lsum_ref[0, 0] = acc_ref[0] * ((1.0 + BETA) / (N_ROWS * DIM))


def _distances_argmin(x4d, embedding):
    return pl.pallas_call(
        _argmin_body,
        grid=(N_BLOCKS,),
        in_specs=[
            pl.BlockSpec((4, 32, 32, DIM), lambda i: (i, 0, 0, 0)),
            pl.BlockSpec((DIM, K_CODES), lambda i: (0, 0)),
        ],
        out_specs=[
            pl.BlockSpec((IDX_ROWS, 128), lambda i: (i, 0)),
            pl.BlockSpec(memory_space=pltpu.SMEM),
        ],
        out_shape=[
            jax.ShapeDtypeStruct((N_CHUNKS, CHUNK), jnp.int32),
            jax.ShapeDtypeStruct((1, 1), jnp.float32),
        ],
        scratch_shapes=[pltpu.SMEM((1,), jnp.float32)],
    )(x4d, embedding)


def _gather_body(table_hbm, idx_hbm, out_hbm, idx_v, rows_v, sem, sem_out):
    wid = lax.axis_index("s") * NC + lax.axis_index("c")
    base = wid * CH_PER_W
    pltpu.sync_copy(idx_hbm.at[pl.ds(base, CH_PER_W)], idx_v)
    copies = [
        pltpu.async_copy(table_hbm.at[idx_v.at[j]], rows_v.at[j], sem)
        for j in range(CH_PER_W)
    ]
    for c in copies:
        c.wait()
    pltpu.sync_copy(rows_v, out_hbm.at[pl.ds(base, CH_PER_W)])


def _sc_gather(table, idx2d):
    k = pl.kernel(
        _gather_body,
        mesh=plsc.VectorSubcoreMesh(core_axis_name="c", subcore_axis_name="s"),
        out_type=jax.ShapeDtypeStruct((N_CHUNKS, CHUNK, DIM), jnp.float32),
        scratch_types=[
            pltpu.VMEM((CH_PER_W, CHUNK), jnp.int32),
            pltpu.VMEM((CH_PER_W, CHUNK, DIM), jnp.float32),
            pltpu.SemaphoreType.DMA,
            pltpu.SemaphoreType.DMA,
        ],
        compiler_params=pltpu.CompilerParams(use_tc_tiling_on_sc=False),
    )
    return k(table, idx2d)


def kernel(x, embedding):
    table = embedding.T
    idx2d, lsum = _distances_argmin(x, embedding)
    q = _sc_gather(table, idx2d)
    quantized = q.reshape(x.shape)
    loss = lsum.reshape(())
    return quantized, loss


# speedup vs baseline: 1.2544x; 1.0200x over previous
import jax
import jax.numpy as jnp
from jax import lax
from jax.experimental import pallas as pl
from jax.experimental.pallas import tpu as pltpu
from jax.experimental.pallas import tpu_sc as plsc

K_CODES = 1024
DIM = 64
BETA = 0.25

N_ROWS = 16384
ROW_BLOCK = 4096
N_BLOCKS = N_ROWS // ROW_BLOCK
IDX_ROWS = ROW_BLOCK // 128

NC = 1
NS = 16
NW = NC * NS
CHUNK = 128
N_CHUNKS = N_ROWS // CHUNK
CH_PER_W = N_CHUNKS // NW


def _argmin_body(x_ref, e_ref, idx_ref, lsum_ref, acc_ref):
    i = pl.program_id(0)
    x = x_ref[...].reshape(ROW_BLOCK, DIM)
    e = e_ref[...]
    pv = jnp.sum(x * x, axis=1, keepdims=True)
    el = jnp.sum(e * e, axis=0, keepdims=True)
    s2 = jnp.dot(x + x, e, preferred_element_type=jnp.float32)
    dist = (pv + el) - s2
    rowmin = jnp.min(dist, axis=1, keepdims=True)
    cols = lax.broadcasted_iota(jnp.int32, dist.shape, 1).astype(jnp.float32)
    idxf = jnp.min(jnp.where(dist == rowmin, cols, float(K_CODES)), axis=1, keepdims=True)
    idx_ref[...] = idxf.astype(jnp.int32).reshape(IDX_ROWS, 128)

    @pl.when(i == 0)
    def _init():
        acc_ref[0] = 0.0

    acc_ref[0] += jnp.sum(rowmin)

    @pl.when(i == N_BLOCKS - 1)
    def _fini():
        lsum_ref[0, 0] = acc_ref[0] * ((1.0 + BETA) / (N_ROWS * DIM))


def _distances_argmin(x4d, embedding):
    return pl.pallas_call(
        _argmin_body,
        grid=(N_BLOCKS,),
        in_specs=[
            pl.BlockSpec((4, 32, 32, DIM), lambda i: (i, 0, 0, 0)),
            pl.BlockSpec((DIM, K_CODES), lambda i: (0, 0)),
        ],
        out_specs=[
            pl.BlockSpec((IDX_ROWS, 128), lambda i: (i, 0)),
            pl.BlockSpec(memory_space=pltpu.SMEM),
        ],
        out_shape=[
            jax.ShapeDtypeStruct((N_CHUNKS, CHUNK), jnp.int32),
            jax.ShapeDtypeStruct((1, 1), jnp.float32),
        ],
        scratch_shapes=[pltpu.SMEM((1,), jnp.float32)],
    )(x4d, embedding)


def _gather_body(table_hbm, idx_hbm, out_hbm, idx_v, rows_v, sem, sem_out):
    wid = lax.axis_index("s") * NC + lax.axis_index("c")
    base = wid * CH_PER_W
    pltpu.sync_copy(idx_hbm.at[pl.ds(base, CH_PER_W)], idx_v)
    copies = [
        pltpu.async_copy(table_hbm.at[idx_v.at[j]], rows_v.at[j], sem)
        for j in range(CH_PER_W)
    ]
    for c in copies:
        c.wait()
    pltpu.sync_copy(rows_v, out_hbm.at[pl.ds(base, CH_PER_W)])


def _sc_gather(table, idx2d):
    k = pl.kernel(
        _gather_body,
        mesh=plsc.VectorSubcoreMesh(core_axis_name="c", subcore_axis_name="s", num_cores=1),
        out_type=jax.ShapeDtypeStruct((N_CHUNKS, CHUNK, DIM), jnp.float32),
        scratch_types=[
            pltpu.VMEM((CH_PER_W, CHUNK), jnp.int32),
            pltpu.VMEM((CH_PER_W, CHUNK, DIM), jnp.float32),
            pltpu.SemaphoreType.DMA,
            pltpu.SemaphoreType.DMA,
        ],
        compiler_params=pltpu.CompilerParams(use_tc_tiling_on_sc=False),
    )
    return k(table, idx2d)


def kernel(x, embedding):
    table = embedding.T
    idx2d, lsum = _distances_argmin(x, embedding)
    q = _sc_gather(table, idx2d)
    quantized = q.reshape(x.shape)
    loss = lsum.reshape(())
    return quantized, loss


# speedup vs baseline: 1.3034x; 1.0390x over previous
import jax
import jax.numpy as jnp
from jax import lax
from jax.experimental import pallas as pl
from jax.experimental.pallas import tpu as pltpu
from jax.experimental.pallas import tpu_sc as plsc

K_CODES = 1024
DIM = 64
BETA = 0.25

N_ROWS = 16384
ROW_BLOCK = 4096
N_BLOCKS = N_ROWS // ROW_BLOCK
IDX_ROWS = ROW_BLOCK // 128
LCHUNK = 256

NC = 1
NS = 16
NW = NC * NS
CHUNK = 128
N_CHUNKS = N_ROWS // CHUNK
CH_PER_W = N_CHUNKS // NW


def _argmin_body(x_ref, e_ref, idx_ref, lsum_ref, acc_ref):
    i = pl.program_id(0)
    x = x_ref[...].reshape(ROW_BLOCK, DIM)
    e = e_ref[...]
    pv = jnp.sum(x * x, axis=1, keepdims=True)
    el = jnp.sum(e * e, axis=0, keepdims=True)
    s2 = jnp.dot(x + x, e, preferred_element_type=jnp.float32)
    cb = lax.broadcasted_iota(jnp.int32, (ROW_BLOCK, LCHUNK), 1).astype(jnp.float32)
    bestv = (pv + el[:, :LCHUNK]) - s2[:, :LCHUNK]
    bestc = cb
    for k in range(1, K_CODES // LCHUNK):
        d = (pv + el[:, k * LCHUNK:(k + 1) * LCHUNK]) - s2[:, k * LCHUNK:(k + 1) * LCHUNK]
        m = d < bestv
        bestv = jnp.where(m, d, bestv)
        bestc = jnp.where(m, cb + float(k * LCHUNK), bestc)
    rowmin = jnp.min(bestv, axis=1, keepdims=True)
    idxf = jnp.min(jnp.where(bestv == rowmin, bestc, float(K_CODES)), axis=1, keepdims=True)
    idx_ref[...] = idxf.astype(jnp.int32).reshape(IDX_ROWS, 128)

    @pl.when(i == 0)
    def _init():
        acc_ref[0] = 0.0

    acc_ref[0] += jnp.sum(rowmin)

    @pl.when(i == N_BLOCKS - 1)
    def _fini():
        lsum_ref[0, 0] = acc_ref[0] * ((1.0 + BETA) / (N_ROWS * DIM))


def _distances_argmin(x4d, embedding):
    return pl.pallas_call(
        _argmin_body,
        grid=(N_BLOCKS,),
        in_specs=[
            pl.BlockSpec((4, 32, 32, DIM), lambda i: (i, 0, 0, 0)),
            pl.BlockSpec((DIM, K_CODES), lambda i: (0, 0)),
        ],
        out_specs=[
            pl.BlockSpec((IDX_ROWS, 128), lambda i: (i, 0)),
            pl.BlockSpec(memory_space=pltpu.SMEM),
        ],
        out_shape=[
            jax.ShapeDtypeStruct((N_CHUNKS, CHUNK), jnp.int32),
            jax.ShapeDtypeStruct((1, 1), jnp.float32),
        ],
        scratch_shapes=[pltpu.SMEM((1,), jnp.float32)],
    )(x4d, embedding)


def _gather_body(table_hbm, idx_hbm, out_hbm, idx_v, rows_v, sem, sem_out):
    wid = lax.axis_index("s") * NC + lax.axis_index("c")
    base = wid * CH_PER_W
    pltpu.sync_copy(idx_hbm.at[pl.ds(base, CH_PER_W)], idx_v)
    copies = [
        pltpu.async_copy(table_hbm.at[idx_v.at[j]], rows_v.at[j], sem)
        for j in range(CH_PER_W)
    ]
    for c in copies:
        c.wait()
    pltpu.sync_copy(rows_v, out_hbm.at[pl.ds(base, CH_PER_W)])


def _sc_gather(table, idx2d):
    k = pl.kernel(
        _gather_body,
        mesh=plsc.VectorSubcoreMesh(core_axis_name="c", subcore_axis_name="s", num_cores=1),
        out_type=jax.ShapeDtypeStruct((N_CHUNKS, CHUNK, DIM), jnp.float32),
        scratch_types=[
            pltpu.VMEM((CH_PER_W, CHUNK), jnp.int32),
            pltpu.VMEM((CH_PER_W, CHUNK, DIM), jnp.float32),
            pltpu.SemaphoreType.DMA,
            pltpu.SemaphoreType.DMA,
        ],
        compiler_params=pltpu.CompilerParams(use_tc_tiling_on_sc=False),
    )
    return k(table, idx2d)


def kernel(x, embedding):
    table = embedding.T
    idx2d, lsum = _distances_argmin(x, embedding)
    q = _sc_gather(table, idx2d)
    quantized = q.reshape(x.shape)
    loss = lsum.reshape(())
    return quantized, loss
